# Initial kernel scaffold; baseline (speedup 1.0000x reference)
import functools

import jax
import jax.numpy as jnp
from jax import lax
from jax.experimental import pallas as pl
from jax.experimental.pallas import tpu as pltpu
from jax.experimental.pallas import tpu_sc as plsc

N = 10000
D = 128
G = 64
E = 320000

NC = 2
NS = 16
NW = NC * NS
CH = 128
CPW = 79
EPAD = NW * CPW * CH
NPAD = 10016
RPT = NPAD // NS

_MESH = plsc.VectorSubcoreMesh(
    core_axis_name="c", subcore_axis_name="s", num_cores=NC, num_subcores=NS
)


@functools.partial(
    pl.kernel,
    out_type=jax.ShapeDtypeStruct((NC, NPAD, 8), jnp.float32),
    mesh=_MESH,
    scratch_types=[
        pltpu.VMEM((CPW, CH), jnp.int32),
        pltpu.VMEM((CH, 8), jnp.float32),
        pltpu.VMEM_SHARED((NPAD, 8), jnp.float32),
    ],
)
def _deg_kernel(dst_hbm, ones_hbm, zeros_hbm, out_hbm, dst_t, ones_t, acc):
    c = lax.axis_index("c")
    s = lax.axis_index("s")
    wid = c * NS + s
    pltpu.sync_copy(zeros_hbm, acc.at[pl.ds(s * RPT, RPT), :])
    pltpu.sync_copy(dst_hbm.at[pl.ds(wid * CPW, CPW), :], dst_t)
    pltpu.sync_copy(ones_hbm, ones_t)
    plsc.subcore_barrier()

    def step(j, carry):
        pltpu.sync_copy(ones_t, acc.at[dst_t.at[j]], add=True)
        return carry

    lax.fori_loop(0, CPW, step, 0)
    plsc.subcore_barrier()
    pltpu.sync_copy(
        acc.at[pl.ds(s * RPT, RPT), :], out_hbm.at[c, pl.ds(s * RPT, RPT), :]
    )


@functools.partial(
    pl.kernel,
    out_type=jax.ShapeDtypeStruct((NC, NPAD, D), jnp.float32),
    mesh=_MESH,
    scratch_types=[
        pltpu.VMEM((CPW, CH), jnp.int32),
        pltpu.VMEM((CPW, CH), jnp.int32),
        pltpu.VMEM((CH, D), jnp.float32),
        pltpu.VMEM_SHARED((NPAD, D), jnp.float32),
        pltpu.SemaphoreType.DMA,
    ],
)
def _edge_sum_kernel(t_hbm, src_hbm, dst_hbm, zeros_hbm, out_hbm,
                     src_t, dst_t, rows, acc, sem):
    c = lax.axis_index("c")
    s = lax.axis_index("s")
    wid = c * NS + s
    pltpu.sync_copy(zeros_hbm, acc.at[pl.ds(s * RPT, RPT), :])
    pltpu.sync_copy(src_hbm.at[pl.ds(wid * CPW, CPW), :], src_t)
    pltpu.sync_copy(dst_hbm.at[pl.ds(wid * CPW, CPW), :], dst_t)
    plsc.subcore_barrier()

    def step(j, carry):
        pltpu.async_copy(t_hbm.at[src_t.at[j]], rows, sem).wait()
        pltpu.sync_copy(rows, acc.at[dst_t.at[j]], add=True)
        return carry

    lax.fori_loop(0, CPW, step, 0)
    plsc.subcore_barrier()
    pltpu.sync_copy(
        acc.at[pl.ds(s * RPT, RPT), :], out_hbm.at[c, pl.ds(s * RPT, RPT), :]
    )


def _dis_from_parts(deg_ref):
    deg8 = deg_ref[0] + deg_ref[1]
    return lax.rsqrt(1.0 + deg8[:N, 0:1])


def _tc_first_body(deg_ref, x_ref, w_ref, t_ref):
    dis = _dis_from_parts(deg_ref)
    t_ref[...] = jnp.dot(x_ref[...], w_ref[...],
                         preferred_element_type=jnp.float32) * dis


def _tc_mid_body(deg_ref, s_ref, t_ref, b_ref, w_ref, out_ref):
    dis = _dis_from_parts(deg_ref)
    ssum = s_ref[0, :N, :] + s_ref[1, :N, :]
    h = jax.nn.relu(dis * (ssum + t_ref[...]) + b_ref[...])
    out_ref[...] = jnp.dot(h, w_ref[...],
                           preferred_element_type=jnp.float32) * dis


def _tc_pool_body(deg_ref, s_ref, t_ref, b_ref, batch_ref, out_ref):
    dis = _dis_from_parts(deg_ref)
    ssum = s_ref[0, :N, :] + s_ref[1, :N, :]
    h = dis * (ssum + t_ref[...]) + b_ref[...]
    gids = lax.broadcasted_iota(jnp.int32, (N, G), 1)
    mask = (batch_ref[...] == gids).astype(jnp.float32)
    sums = lax.dot_general(mask, h, (((0,), (0,)), ((), ())),
                           preferred_element_type=jnp.float32)
    counts = jnp.sum(mask, axis=0)
    out_ref[...] = sums / jnp.maximum(counts, 1.0)[:, None]


_tc_first = pl.pallas_call(
    _tc_first_body, out_shape=jax.ShapeDtypeStruct((N, D), jnp.float32))
_tc_mid = pl.pallas_call(
    _tc_mid_body, out_shape=jax.ShapeDtypeStruct((N, D), jnp.float32))
_tc_pool = pl.pallas_call(
    _tc_pool_body, out_shape=jax.ShapeDtypeStruct((G, D), jnp.float32))


@jax.jit
def kernel(x, edge_index, batch, W1, b1, W2, b2, W3, b3):
    pad = EPAD - E
    src = jnp.concatenate(
        [edge_index[0], jnp.zeros((pad,), jnp.int32)]).reshape(NW * CPW, CH)
    dst = jnp.concatenate(
        [edge_index[1], jnp.full((pad,), N, jnp.int32)]).reshape(NW * CPW, CH)

    ones8 = jnp.ones((CH, 8), jnp.float32)
    zeros8 = jnp.zeros((RPT, 8), jnp.float32)
    zerosd = jnp.zeros((RPT, D), jnp.float32)
    batch2d = batch.reshape(N, 1)

    deg_parts = _deg_kernel(dst, ones8, zeros8)

    t1 = _tc_first(deg_parts, x, W1)
    s1 = _edge_sum_kernel(t1, src, dst, zerosd)
    t2 = _tc_mid(deg_parts, s1, t1, b1, W2)
    s2 = _edge_sum_kernel(t2, src, dst, zerosd)
    t3 = _tc_mid(deg_parts, s2, t2, b2, W3)
    s3 = _edge_sum_kernel(t3, src, dst, zerosd)
    return _tc_pool(deg_parts, s3, t3, b3, batch2d)


# speedup vs baseline: 6.6851x; 6.6851x over previous
import functools

import jax
import jax.numpy as jnp
from jax import lax
from jax.experimental import pallas as pl
from jax.experimental.pallas import tpu as pltpu
from jax.experimental.pallas import tpu_sc as plsc

N = 10000
D = 128
G = 64
E = 320000

NC = 2
NS = 16
NW = NC * NS
CH = 128
CPW = 80
EPAD = NW * CPW * CH
NPAD = 10112
RPT = NPAD // NS

_MESH = plsc.VectorSubcoreMesh(
    core_axis_name="c", subcore_axis_name="s", num_cores=NC, num_subcores=NS
)


@functools.partial(
    pl.kernel,
    out_type=jax.ShapeDtypeStruct((NC, NPAD, D), jnp.float32),
    mesh=_MESH,
    scratch_types=[
        pltpu.VMEM((CPW, CH), jnp.int32),
        pltpu.VMEM((CH, D), jnp.float32),
        pltpu.VMEM_SHARED((NPAD, D), jnp.float32),
    ],
)
def _deg_kernel(dst_hbm, ones_hbm, zeros_hbm, out_hbm, dst_t, ones_t, acc):
    c = lax.axis_index("c")
    s = lax.axis_index("s")
    wid = c * NS + s
    pltpu.sync_copy(zeros_hbm, acc.at[pl.ds(s * RPT, RPT), :])
    pltpu.sync_copy(dst_hbm.at[pl.ds(wid * CPW, CPW), :], dst_t)
    pltpu.sync_copy(ones_hbm, ones_t)
    plsc.subcore_barrier()

    def step(j, carry):
        pltpu.sync_copy(ones_t, acc.at[dst_t.at[j]], add=True)
        return carry

    lax.fori_loop(0, CPW, step, 0)
    plsc.subcore_barrier()
    pltpu.sync_copy(
        acc.at[pl.ds(s * RPT, RPT), :], out_hbm.at[c, pl.ds(s * RPT, RPT), :]
    )


@functools.partial(
    pl.kernel,
    out_type=jax.ShapeDtypeStruct((NC, NPAD, D), jnp.float32),
    mesh=_MESH,
    scratch_types=[
        pltpu.VMEM((CPW, CH), jnp.int32),
        pltpu.VMEM((CPW, CH), jnp.int32),
        pltpu.VMEM((CH, D), jnp.float32),
        pltpu.VMEM_SHARED((NPAD, D), jnp.float32),
        pltpu.SemaphoreType.DMA,
    ],
)
def _edge_sum_kernel(t_hbm, src_hbm, dst_hbm, zeros_hbm, out_hbm,
                     src_t, dst_t, rows, acc, sem):
    c = lax.axis_index("c")
    s = lax.axis_index("s")
    wid = c * NS + s
    pltpu.sync_copy(zeros_hbm, acc.at[pl.ds(s * RPT, RPT), :])
    pltpu.sync_copy(src_hbm.at[pl.ds(wid * CPW, CPW), :], src_t)
    pltpu.sync_copy(dst_hbm.at[pl.ds(wid * CPW, CPW), :], dst_t)
    plsc.subcore_barrier()

    def step(j, carry):
        pltpu.async_copy(t_hbm.at[src_t.at[j]], rows, sem).wait()
        pltpu.sync_copy(rows, acc.at[dst_t.at[j]], add=True)
        return carry

    lax.fori_loop(0, CPW, step, 0)
    plsc.subcore_barrier()
    pltpu.sync_copy(
        acc.at[pl.ds(s * RPT, RPT), :], out_hbm.at[c, pl.ds(s * RPT, RPT), :]
    )


def _dis_from_parts(deg_ref):
    deg8 = deg_ref[0] + deg_ref[1]
    return lax.rsqrt(1.0 + deg8[:N, 0:1])


def _tc_first_body(deg_ref, x_ref, w_ref, t_ref):
    dis = _dis_from_parts(deg_ref)
    t_ref[...] = jnp.dot(x_ref[...], w_ref[...],
                         preferred_element_type=jnp.float32) * dis


def _tc_mid_body(deg_ref, s_ref, t_ref, b_ref, w_ref, out_ref):
    dis = _dis_from_parts(deg_ref)
    ssum = s_ref[0, :N, :] + s_ref[1, :N, :]
    h = jax.nn.relu(dis * (ssum + t_ref[...]) + b_ref[...])
    out_ref[...] = jnp.dot(h, w_ref[...],
                           preferred_element_type=jnp.float32) * dis


def _tc_pool_body(deg_ref, s_ref, t_ref, b_ref, batch_ref, out_ref):
    dis = _dis_from_parts(deg_ref)
    ssum = s_ref[0, :N, :] + s_ref[1, :N, :]
    h = dis * (ssum + t_ref[...]) + b_ref[...]
    gids = lax.broadcasted_iota(jnp.int32, (N, G), 1)
    mask = (batch_ref[...] == gids).astype(jnp.float32)
    sums = lax.dot_general(mask, h, (((0,), (0,)), ((), ())),
                           preferred_element_type=jnp.float32)
    counts = jnp.sum(mask, axis=0)
    out_ref[...] = sums / jnp.maximum(counts, 1.0)[:, None]


_tc_first = pl.pallas_call(
    _tc_first_body, out_shape=jax.ShapeDtypeStruct((N, D), jnp.float32))
_tc_mid = pl.pallas_call(
    _tc_mid_body, out_shape=jax.ShapeDtypeStruct((N, D), jnp.float32))
_tc_pool = pl.pallas_call(
    _tc_pool_body, out_shape=jax.ShapeDtypeStruct((G, D), jnp.float32))


@jax.jit
def kernel(x, edge_index, batch, W1, b1, W2, b2, W3, b3):
    pad = EPAD - E
    src = jnp.concatenate(
        [edge_index[0], jnp.zeros((pad,), jnp.int32)]).reshape(NW * CPW, CH)
    dst = jnp.concatenate(
        [edge_index[1], jnp.full((pad,), N, jnp.int32)]).reshape(NW * CPW, CH)

    onesd = jnp.ones((CH, D), jnp.float32)
    zerosd = jnp.zeros((RPT, D), jnp.float32)
    batch2d = batch.reshape(N, 1)

    deg_parts = _deg_kernel(dst, onesd, zerosd)

    t1 = _tc_first(deg_parts, x, W1)
    s1 = _edge_sum_kernel(t1, src, dst, zerosd)
    t2 = _tc_mid(deg_parts, s1, t1, b1, W2)
    s2 = _edge_sum_kernel(t2, src, dst, zerosd)
    t3 = _tc_mid(deg_parts, s2, t2, b2, W3)
    s3 = _edge_sum_kernel(t3, src, dst, zerosd)
    return _tc_pool(deg_parts, s3, t3, b3, batch2d)
